# baseline (speedup 1.0000x reference)
import functools

import jax
import jax.numpy as jnp
from jax import lax
from jax.experimental import pallas as pl
from jax.experimental.pallas import tpu as pltpu
from jax.experimental.pallas import tpu_sc as plsc

TOK_DIM = 57
STRUCT_DIM = 7
D_MODEL = 64

_info = plsc.get_sparse_core_info()
NC, NS, NLANES = _info.num_cores, _info.num_subcores, _info.num_lanes
NW = NC * NS

CHUNK = 128


def _sc_embed(BL):
    per_w = BL // NW
    n_chunks = per_w // CHUNK
    mesh = plsc.VectorSubcoreMesh(core_axis_name="c", subcore_axis_name="s")

    @functools.partial(
        pl.kernel,
        mesh=mesh,
        compiler_params=pltpu.CompilerParams(
            needs_layout_passes=False, use_tc_tiling_on_sc=False),
        out_type=jax.ShapeDtypeStruct((BL, D_MODEL), jnp.float32),
        scratch_types=[
            pltpu.VMEM((CHUNK,), jnp.int32),
            pltpu.VMEM((CHUNK, D_MODEL), jnp.float32),
            pltpu.VMEM((CHUNK * STRUCT_DIM,), jnp.float32),
            pltpu.VMEM((7 * 7 + 7, NLANES), jnp.float32),
            pltpu.SemaphoreType.DMA,
        ],
    )
    def k(tok_hbm, x_hbm, table_hbm, wb_hbm, out_hbm,
          idx_v, out_v, x_v, wb_v, sem):
        wid = lax.axis_index("s") * NC + lax.axis_index("c")
        wbase = wid * per_w
        pltpu.sync_copy(wb_hbm, wb_v)

        def body(c, _):
            base = wbase + c * CHUNK
            pltpu.sync_copy(tok_hbm.at[pl.ds(base, CHUNK)], idx_v)
            pltpu.async_copy(table_hbm.at[idx_v], out_v, sem).wait()
            pltpu.sync_copy(
                x_hbm.at[pl.ds(base * STRUCT_DIM, CHUNK * STRUCT_DIM)], x_v)
            for g in range(CHUNK // NLANES):
                ridx = g * NLANES + lax.iota(jnp.int32, NLANES)
                fidx = ridx * STRUCT_DIM
                xd = [plsc.load_gather(x_v, [fidx + d])
                      for d in range(STRUCT_DIM)]
                for e in range(STRUCT_DIM):
                    acc = wb_v[49 + e, :]
                    for d in range(STRUCT_DIM):
                        acc = acc + xd[d] * wb_v[e * STRUCT_DIM + d, :]
                    plsc.store_scatter(
                        out_v,
                        [ridx, jnp.full((NLANES,), TOK_DIM + e, jnp.int32)],
                        acc)
            pltpu.sync_copy(out_v, out_hbm.at[pl.ds(base, CHUNK)])
            return ()

        lax.fori_loop(0, n_chunks, body, ())

    return k


def kernel(tokens, structure_features, table, W, b):
    B, L = tokens.shape
    BL = B * L
    tok_flat = tokens.reshape(BL).astype(jnp.int32)
    x_flat = structure_features.reshape(BL * STRUCT_DIM)
    wb = jnp.broadcast_to(
        jnp.concatenate([W.reshape(-1), b], axis=0)[:, None], (56, NLANES))
    table64 = jnp.pad(table, ((0, 0), (0, D_MODEL - TOK_DIM)))
    out = _sc_embed(BL)(tok_flat, x_flat, table64, wb)
    return out.reshape(B, L, D_MODEL)
